# Initial kernel scaffold; baseline (speedup 1.0000x reference)
import functools

import jax
import jax.numpy as jnp
from jax import lax
from jax.experimental import pallas as pl
from jax.experimental.pallas import tpu as pltpu
from jax.experimental.pallas import tpu_sc as plsc

N = 10000
NP = 10240
E = 320000
CH = 128
NWORK = 32
CPW = 80
EP = NWORK * CPW * CH
NCH = EP // CH
RPT = NP // 16
BLK = 1024
GRID = NP // BLK


def _make_sc_segsum(D):
  mesh = plsc.VectorSubcoreMesh(core_axis_name="c", subcore_axis_name="s")

  @functools.partial(
      pl.kernel,
      mesh=mesh,
      out_type=jax.ShapeDtypeStruct((2, NP, D), jnp.float32),
      scratch_types=[
          pltpu.VMEM((CPW, CH), jnp.int32),
          pltpu.VMEM((CPW, CH), jnp.int32),
          pltpu.VMEM((CH, D), jnp.float32),
          pltpu.VMEM_SHARED((NP, D), jnp.float32),
          pltpu.SemaphoreType.DMA,
      ],
  )
  def seg(table, srcg, dstg, out, src_v, dst_v, rows_v, acc, sem):
    cid = lax.axis_index("c")
    sid = lax.axis_index("s")
    wid = sid * 2 + cid

    def zrow(r, _):
      def zcol(c, _):
        rows_v[r, pl.ds(c * 16, 16)] = jnp.zeros((16,), jnp.float32)
        return 0
      return lax.fori_loop(0, D // 16, zcol, 0)
    lax.fori_loop(0, CH, zrow, 0)
    for k in range(RPT // CH):
      pltpu.sync_copy(rows_v, acc.at[pl.ds(sid * RPT + k * CH, CH)])
    plsc.subcore_barrier()

    pltpu.sync_copy(srcg.at[pl.ds(wid * CPW, CPW)], src_v)
    pltpu.sync_copy(dstg.at[pl.ds(wid * CPW, CPW)], dst_v)

    def body(j, _):
      pltpu.async_copy(table.at[src_v.at[j]], rows_v, sem).wait()
      pltpu.sync_copy(rows_v, acc.at[dst_v.at[j]], add=True)
      return 0
    lax.fori_loop(0, CPW, body, 0)

    plsc.subcore_barrier()
    pltpu.sync_copy(acc.at[pl.ds(sid * RPT, RPT)],
                    out.at[cid, pl.ds(sid * RPT, RPT)])

  return seg


_sc_seg144 = _make_sc_segsum(144)
_sc_seg128 = _make_sc_segsum(128)
_sc_seg48 = _make_sc_segsum(48)


def _l0_body(x_ref, p0_ref, p1_ref, ws_ref, wn_ref, b_ref, h_ref, inv_ref):
  s = p0_ref[:, :128] + p1_ref[:, :128]
  deg = p0_ref[:, 128:129] + p1_ref[:, 128:129]
  inv = 1.0 / jnp.maximum(deg, 1.0)
  inv_ref[...] = jnp.broadcast_to(inv, (BLK, 128))
  hn = s * inv
  h = jnp.dot(x_ref[...], ws_ref[...], preferred_element_type=jnp.float32)
  h = h + jnp.dot(hn, wn_ref[...], preferred_element_type=jnp.float32)
  h = h + b_ref[...]
  h_ref[...] = jnp.maximum(h, 0.0)


def _layer0(xp, p0, p1, Ws0, Wn0, b0):
  return pl.pallas_call(
      _l0_body,
      grid=(GRID,),
      in_specs=[
          pl.BlockSpec((BLK, 128), lambda i: (i, 0)),
          pl.BlockSpec((BLK, 144), lambda i: (i, 0)),
          pl.BlockSpec((BLK, 144), lambda i: (i, 0)),
          pl.BlockSpec((128, 256), lambda i: (0, 0)),
          pl.BlockSpec((128, 256), lambda i: (0, 0)),
          pl.BlockSpec((1, 256), lambda i: (0, 0)),
      ],
      out_specs=[
          pl.BlockSpec((BLK, 256), lambda i: (i, 0)),
          pl.BlockSpec((BLK, 128), lambda i: (i, 0)),
      ],
      out_shape=[
          jax.ShapeDtypeStruct((NP, 256), jnp.float32),
          jax.ShapeDtypeStruct((NP, 128), jnp.float32),
      ],
  )(xp, p0, p1, Ws0, Wn0, b0)


def _l1_body(h1_ref, qlo0_ref, qlo1_ref, qhi0_ref, qhi1_ref, inv_ref,
             ws1_ref, wn1a_ref, wn1b_ref, b1_ref, ws2_ref, wn2_ref, b2_ref,
             m2_ref, z_ref):
  inv = inv_ref[:, :1]
  slo = (qlo0_ref[...] + qlo1_ref[...]) * inv
  shi = (qhi0_ref[...] + qhi1_ref[...]) * inv
  h = jnp.dot(h1_ref[...], ws1_ref[...], preferred_element_type=jnp.float32)
  h = h + jnp.dot(slo, wn1a_ref[...], preferred_element_type=jnp.float32)
  h = h + jnp.dot(shi, wn1b_ref[...], preferred_element_type=jnp.float32)
  h = jnp.maximum(h + b1_ref[...], 0.0)
  m2_ref[...] = jnp.dot(h, wn2_ref[...], preferred_element_type=jnp.float32)
  z_ref[...] = jnp.dot(h, ws2_ref[...],
                       preferred_element_type=jnp.float32) + b2_ref[...]


def _layer1(h1, qlo0, qlo1, qhi0, qhi1, inv2d, Ws1, Wn1a, Wn1b, b1,
            Ws2p, Wn2p, b2p):
  return pl.pallas_call(
      _l1_body,
      grid=(GRID,),
      in_specs=[
          pl.BlockSpec((BLK, 256), lambda i: (i, 0)),
          pl.BlockSpec((BLK, 128), lambda i: (i, 0)),
          pl.BlockSpec((BLK, 128), lambda i: (i, 0)),
          pl.BlockSpec((BLK, 128), lambda i: (i, 0)),
          pl.BlockSpec((BLK, 128), lambda i: (i, 0)),
          pl.BlockSpec((BLK, 128), lambda i: (i, 0)),
          pl.BlockSpec((256, 256), lambda i: (0, 0)),
          pl.BlockSpec((128, 256), lambda i: (0, 0)),
          pl.BlockSpec((128, 256), lambda i: (0, 0)),
          pl.BlockSpec((1, 256), lambda i: (0, 0)),
          pl.BlockSpec((256, 48), lambda i: (0, 0)),
          pl.BlockSpec((256, 48), lambda i: (0, 0)),
          pl.BlockSpec((1, 48), lambda i: (0, 0)),
      ],
      out_specs=[
          pl.BlockSpec((BLK, 48), lambda i: (i, 0)),
          pl.BlockSpec((BLK, 48), lambda i: (i, 0)),
      ],
      out_shape=[
          jax.ShapeDtypeStruct((NP, 48), jnp.float32),
          jax.ShapeDtypeStruct((NP, 48), jnp.float32),
      ],
  )(h1, qlo0, qlo1, qhi0, qhi1, inv2d, Ws1, Wn1a, Wn1b, b1, Ws2p, Wn2p, b2p)


def _fin_body(z_ref, r0_ref, r1_ref, inv_ref, out_ref):
  s = (r0_ref[:N, :47] + r1_ref[:N, :47]) * inv_ref[:N, :1]
  out_ref[...] = z_ref[:N, :47] + s


def _final(z, r0, r1, inv2d):
  return pl.pallas_call(
      _fin_body,
      grid=(1,),
      in_specs=[
          pl.BlockSpec((NP, 48), lambda i: (0, 0)),
          pl.BlockSpec((NP, 48), lambda i: (0, 0)),
          pl.BlockSpec((NP, 48), lambda i: (0, 0)),
          pl.BlockSpec((NP, 128), lambda i: (0, 0)),
      ],
      out_specs=pl.BlockSpec((N, 47), lambda i: (0, 0)),
      out_shape=jax.ShapeDtypeStruct((N, 47), jnp.float32),
  )(z, r0, r1, inv2d)


def kernel(x, edge_index, W_self0, W_neigh0, b0, W_self1, W_neigh1, b1,
           W_self2, W_neigh2, b2):
  src = edge_index[0]
  dst = edge_index[1]
  srcg = jnp.concatenate(
      [src, jnp.zeros((EP - E,), jnp.int32)]).reshape(NCH, CH)
  dstg = jnp.concatenate(
      [dst, jnp.full((EP - E,), N, jnp.int32)]).reshape(NCH, CH)

  xp = jnp.pad(x, ((0, NP - N), (0, 0)))
  table0 = jnp.concatenate([xp, jnp.ones((NP, 16), jnp.float32)], axis=1)

  p = _sc_seg144(table0, srcg, dstg)
  h1, inv2d = _layer0(xp, p[0], p[1], W_self0, W_neigh0,
                      b0.reshape(1, 256))

  qlo = _sc_seg128(h1[:, :128], srcg, dstg)
  qhi = _sc_seg128(h1[:, 128:], srcg, dstg)

  Ws2p = jnp.pad(W_self2, ((0, 0), (0, 1)))
  Wn2p = jnp.pad(W_neigh2, ((0, 0), (0, 1)))
  b2p = jnp.pad(b2, ((0, 1),)).reshape(1, 48)
  m2, z = _layer1(h1, qlo[0], qlo[1], qhi[0], qhi[1], inv2d,
                  W_self1, W_neigh1[:128], W_neigh1[128:],
                  b1.reshape(1, 256), Ws2p, Wn2p, b2p)

  r = _sc_seg48(m2, srcg, dstg)
  return _final(z, r[0], r[1], inv2d)


# speedup vs baseline: 3.3866x; 3.3866x over previous
import functools

import jax
import jax.numpy as jnp
from jax import lax
from jax.experimental import pallas as pl
from jax.experimental.pallas import tpu as pltpu
from jax.experimental.pallas import tpu_sc as plsc

N = 10000
NP = 10240
E = 320000
CH = 128
NWORK = 32
CPW = 80
EP = NWORK * CPW * CH
NCH = EP // CH
RPT = NP // 16
BLK = 1024
GRID = NP // BLK


_SC_CACHE = {}


def _make_sc_segsum(D):
  if D in _SC_CACHE:
    return _SC_CACHE[D]
  mesh = plsc.VectorSubcoreMesh(core_axis_name="c", subcore_axis_name="s")

  @functools.partial(
      pl.kernel,
      mesh=mesh,
      compiler_params=pltpu.CompilerParams(use_tc_tiling_on_sc=False),
      out_type=jax.ShapeDtypeStruct((2, NP, D), jnp.float32),
      scratch_types=[
          pltpu.VMEM((CPW, CH), jnp.int32),
          pltpu.VMEM((CPW, CH), jnp.int32),
          pltpu.VMEM((CH, D), jnp.float32),
          pltpu.VMEM_SHARED((NP, D), jnp.float32),
          pltpu.SemaphoreType.DMA,
      ],
  )
  def seg(table, srcg, dstg, out, src_v, dst_v, rows_v, acc, sem):
    cid = lax.axis_index("c")
    sid = lax.axis_index("s")
    wid = sid * 2 + cid

    def zrow(r, _):
      def zcol(c, _):
        rows_v[r, pl.ds(c * 16, 16)] = jnp.zeros((16,), jnp.float32)
        return 0
      return lax.fori_loop(0, D // 16, zcol, 0)
    lax.fori_loop(0, CH, zrow, 0)
    for k in range(RPT // CH):
      pltpu.sync_copy(rows_v, acc.at[pl.ds(sid * RPT + k * CH, CH)])
    plsc.subcore_barrier()

    pltpu.sync_copy(srcg.at[pl.ds(wid * CPW, CPW)], src_v)
    pltpu.sync_copy(dstg.at[pl.ds(wid * CPW, CPW)], dst_v)

    def body(j, _):
      pltpu.async_copy(table.at[src_v.at[j]], rows_v, sem).wait()
      pltpu.sync_copy(rows_v, acc.at[dst_v.at[j]], add=True)
      return 0
    lax.fori_loop(0, CPW, body, 0)

    plsc.subcore_barrier()
    pltpu.sync_copy(acc.at[pl.ds(sid * RPT, RPT)],
                    out.at[cid, pl.ds(sid * RPT, RPT)])

  _SC_CACHE[D] = seg
  return seg


def _sc_segsum(table, srcg, dstg):
  return _make_sc_segsum(table.shape[1])(table, srcg, dstg)


def _l0_body(x_ref, p0_ref, p1_ref, ws_ref, wn_ref, b_ref, h_ref, inv_ref):
  s = p0_ref[:, :128] + p1_ref[:, :128]
  deg = p0_ref[:, 128:129] + p1_ref[:, 128:129]
  inv = 1.0 / jnp.maximum(deg, 1.0)
  inv_ref[...] = jnp.broadcast_to(inv, (BLK, 128))
  hn = s * inv
  h = jnp.dot(x_ref[...], ws_ref[...], preferred_element_type=jnp.float32)
  h = h + jnp.dot(hn, wn_ref[...], preferred_element_type=jnp.float32)
  h = h + b_ref[...]
  h_ref[...] = jnp.maximum(h, 0.0)


def _layer0(xp, p0, p1, Ws0, Wn0, b0):
  return pl.pallas_call(
      _l0_body,
      grid=(GRID,),
      in_specs=[
          pl.BlockSpec((BLK, 128), lambda i: (i, 0)),
          pl.BlockSpec((BLK, 144), lambda i: (i, 0)),
          pl.BlockSpec((BLK, 144), lambda i: (i, 0)),
          pl.BlockSpec((128, 256), lambda i: (0, 0)),
          pl.BlockSpec((128, 256), lambda i: (0, 0)),
          pl.BlockSpec((1, 256), lambda i: (0, 0)),
      ],
      out_specs=[
          pl.BlockSpec((BLK, 256), lambda i: (i, 0)),
          pl.BlockSpec((BLK, 128), lambda i: (i, 0)),
      ],
      out_shape=[
          jax.ShapeDtypeStruct((NP, 256), jnp.float32),
          jax.ShapeDtypeStruct((NP, 128), jnp.float32),
      ],
  )(xp, p0, p1, Ws0, Wn0, b0)


def _l1_body(h1_ref, qlo0_ref, qlo1_ref, qhi0_ref, qhi1_ref, inv_ref,
             ws1_ref, wn1a_ref, wn1b_ref, b1_ref, ws2_ref, wn2_ref, b2_ref,
             m2_ref, z_ref):
  inv = inv_ref[:, :1]
  slo = (qlo0_ref[...] + qlo1_ref[...]) * inv
  shi = (qhi0_ref[...] + qhi1_ref[...]) * inv
  h = jnp.dot(h1_ref[...], ws1_ref[...], preferred_element_type=jnp.float32)
  h = h + jnp.dot(slo, wn1a_ref[...], preferred_element_type=jnp.float32)
  h = h + jnp.dot(shi, wn1b_ref[...], preferred_element_type=jnp.float32)
  h = jnp.maximum(h + b1_ref[...], 0.0)
  m2_ref[...] = jnp.dot(h, wn2_ref[...], preferred_element_type=jnp.float32)
  z_ref[...] = jnp.dot(h, ws2_ref[...],
                       preferred_element_type=jnp.float32) + b2_ref[...]


def _layer1(h1, qlo0, qlo1, qhi0, qhi1, inv2d, Ws1, Wn1a, Wn1b, b1,
            Ws2p, Wn2p, b2p):
  return pl.pallas_call(
      _l1_body,
      grid=(GRID,),
      in_specs=[
          pl.BlockSpec((BLK, 256), lambda i: (i, 0)),
          pl.BlockSpec((BLK, 128), lambda i: (i, 0)),
          pl.BlockSpec((BLK, 128), lambda i: (i, 0)),
          pl.BlockSpec((BLK, 128), lambda i: (i, 0)),
          pl.BlockSpec((BLK, 128), lambda i: (i, 0)),
          pl.BlockSpec((BLK, 128), lambda i: (i, 0)),
          pl.BlockSpec((256, 256), lambda i: (0, 0)),
          pl.BlockSpec((128, 256), lambda i: (0, 0)),
          pl.BlockSpec((128, 256), lambda i: (0, 0)),
          pl.BlockSpec((1, 256), lambda i: (0, 0)),
          pl.BlockSpec((256, 48), lambda i: (0, 0)),
          pl.BlockSpec((256, 48), lambda i: (0, 0)),
          pl.BlockSpec((1, 48), lambda i: (0, 0)),
      ],
      out_specs=[
          pl.BlockSpec((BLK, 48), lambda i: (i, 0)),
          pl.BlockSpec((BLK, 48), lambda i: (i, 0)),
      ],
      out_shape=[
          jax.ShapeDtypeStruct((NP, 48), jnp.float32),
          jax.ShapeDtypeStruct((NP, 48), jnp.float32),
      ],
  )(h1, qlo0, qlo1, qhi0, qhi1, inv2d, Ws1, Wn1a, Wn1b, b1, Ws2p, Wn2p, b2p)


def _fin_body(z_ref, r0_ref, r1_ref, inv_ref, out_ref):
  s = (r0_ref[:N, :47] + r1_ref[:N, :47]) * inv_ref[:N, :1]
  out_ref[...] = z_ref[:N, :47] + s


def _final(z, r0, r1, inv2d):
  return pl.pallas_call(
      _fin_body,
      grid=(1,),
      in_specs=[
          pl.BlockSpec((NP, 48), lambda i: (0, 0)),
          pl.BlockSpec((NP, 48), lambda i: (0, 0)),
          pl.BlockSpec((NP, 48), lambda i: (0, 0)),
          pl.BlockSpec((NP, 128), lambda i: (0, 0)),
      ],
      out_specs=pl.BlockSpec((N, 47), lambda i: (0, 0)),
      out_shape=jax.ShapeDtypeStruct((N, 47), jnp.float32),
  )(z, r0, r1, inv2d)


def kernel(x, edge_index, W_self0, W_neigh0, b0, W_self1, W_neigh1, b1,
           W_self2, W_neigh2, b2):
  src = edge_index[0]
  dst = edge_index[1]
  srcg = jnp.concatenate(
      [src, jnp.zeros((EP - E,), jnp.int32)]).reshape(NCH, CH)
  dstg = jnp.concatenate(
      [dst, jnp.full((EP - E,), N, jnp.int32)]).reshape(NCH, CH)

  xp = jnp.pad(x, ((0, NP - N), (0, 0)))
  table0 = jnp.concatenate([xp, jnp.ones((NP, 16), jnp.float32)], axis=1)

  p = _sc_segsum(table0, srcg, dstg)
  h1, inv2d = _layer0(xp, p[0], p[1], W_self0, W_neigh0,
                      b0.reshape(1, 256))

  qlo = _sc_segsum(h1[:, :128], srcg, dstg)
  qhi = _sc_segsum(h1[:, 128:], srcg, dstg)

  Ws2p = jnp.pad(W_self2, ((0, 0), (0, 1)))
  Wn2p = jnp.pad(W_neigh2, ((0, 0), (0, 1)))
  b2p = jnp.pad(b2, ((0, 1),)).reshape(1, 48)
  m2, z = _layer1(h1, qlo[0], qlo[1], qhi[0], qhi[1], inv2d,
                  W_self1, W_neigh1[:128], W_neigh1[128:],
                  b1.reshape(1, 256), Ws2p, Wn2p, b2p)

  r = _sc_segsum(m2, srcg, dstg)
  return _final(z, r[0], r[1], inv2d)
